# baseline (speedup 1.0000x reference)
import dataclasses

import jax
import jax.numpy as jnp
from jax import lax
from jax.experimental import pallas as pl
from jax.experimental.pallas import tpu as pltpu
from jax.experimental.pallas import tpu_sc as plsc

_VOCAB = 1000000
_D = 64
_B = 4096
_C = 20
_NNEG = 20
_R = _C + _NNEG
_NW = 32
_BPW = _B // _NW
_RPW = _BPW * _R
_CB = 8
_CROWS = _CB * _R
_GCH = 80
_NG = _CROWS // _GCH
_NCHUNK = _BPW // _CB
_PCB = 8192
_IVR = 8


def _sc_mesh_params():
    mesh = plsc.VectorSubcoreMesh(core_axis_name="c", subcore_axis_name="s")
    cp = pltpu.CompilerParams(use_tc_tiling_on_sc=True)
    if "needs_layout_passes" in pltpu.CompilerParams.__dataclass_fields__:
        cp = dataclasses.replace(cp, needs_layout_passes=False)
    return mesh, cp


def _sc_ivec(iv_t, iwords):
    mesh, cp = _sc_mesh_params()

    @pl.kernel(
        compiler_params=cp,
        out_type=jax.ShapeDtypeStruct((_B, 128), jnp.float32),
        mesh=mesh,
        scratch_types=[
            pltpu.VMEM((_BPW + 16,), jnp.int32),
            pltpu.VMEM((_IVR, _D, 128), jnp.float32),
            pltpu.VMEM((_BPW, 128), jnp.float32),
            pltpu.SemaphoreType.DMA,
        ],
    )
    def body(ivt_hbm, iw_hbm, out_hbm, iw_v, ivblk_v, ivec_v, csem):
        wid = lax.axis_index("s") * 2 + lax.axis_index("c")
        b0 = pl.multiple_of(wid * _BPW, 8)
        iota16 = lax.iota(jnp.int32, 16)
        pltpu.sync_copy(iw_hbm.at[pl.ds(b0, _BPW)], iw_v.at[pl.ds(0, _BPW)])

        def fire(i, slot):
            w = iw_v[pl.ds(i, 16)][0]
            walign = pl.multiple_of((w // 128) * 128, 128)
            pltpu.async_copy(ivt_hbm.at[:, pl.ds(walign, 128)],
                             ivblk_v.at[slot], csem)

        for i in range(_IVR):
            fire(i, i)

        @pl.loop(0, _BPW)
        def _(i):
            slot = lax.rem(i, _IVR)
            pltpu.make_async_copy(ivt_hbm.at[:, pl.ds(0, 128)],
                                  ivblk_v.at[0], csem).wait()
            wv = iw_v[pl.ds(i, 16)][0]
            lane = jnp.full((16,), wv % 128, jnp.int32)
            for j in range(4):
                dv = j * 16 + iota16
                ivec_v[i, pl.ds(j * 16, 16)] = plsc.load_gather(
                    ivblk_v.at[slot], [dv, lane])

            @pl.when(i + _IVR < _BPW)
            def _():
                fire(i + _IVR, slot)

        pltpu.sync_copy(ivec_v, out_hbm.at[pl.ds(b0, _BPW), :])

    return body(iv_t, iwords)


def _tc_pack(ov_t):
    def body(b_ref, ob_ref):
        ipad = jnp.concatenate(
            [jnp.eye(_D, dtype=jnp.float32),
             jnp.zeros((_D, 128 - _D), jnp.float32)], axis=1)
        ob_ref[...] = lax.dot_general(
            b_ref[...], ipad, (((0,), (0,)), ((), ())),
            precision=lax.Precision.DEFAULT)

    return pl.pallas_call(
        body,
        grid=(pl.cdiv(_VOCAB, _PCB),),
        in_specs=[pl.BlockSpec((_D, _PCB), lambda i: (0, i))],
        out_specs=pl.BlockSpec((_PCB, 128), lambda i: (i, 0)),
        out_shape=jax.ShapeDtypeStruct((_VOCAB, 128), jnp.float32),
    )(ov_t)


def _sc_scores(ovp, ivec, ow_t, nw_t):
    mesh, cp = _sc_mesh_params()

    @pl.kernel(
        compiler_params=cp,
        out_type=jax.ShapeDtypeStruct((_B * _R,), jnp.float32),
        mesh=mesh,
        scratch_types=[
            pltpu.VMEM((_BPW, 128), jnp.float32),
            pltpu.VMEM((_R, _BPW), jnp.int32),
            pltpu.VMEM((_RPW,), jnp.int32),
            pltpu.VMEM((2, _CROWS, 128), jnp.float32),
            pltpu.VMEM((_RPW,), jnp.float32),
            pltpu.SemaphoreType.DMA,
            pltpu.SemaphoreType.DMA,
        ],
    )
    def body(ovp_hbm, ivec_hbm, ow_hbm, nw_hbm, out_hbm,
             ivec_v, words_v, gidx_v, rbuf_v, sc_v, sem_a, sem_b):
        wid = lax.axis_index("s") * 2 + lax.axis_index("c")
        b0 = pl.multiple_of(wid * _BPW, 8)
        r0 = pl.multiple_of(wid * _RPW, 8)
        iota16 = lax.iota(jnp.int32, 16)
        lane0 = iota16 == 0
        pltpu.sync_copy(ow_hbm.at[:, pl.ds(b0, _BPW)],
                        words_v.at[pl.ds(0, _C), :])
        pltpu.sync_copy(nw_hbm.at[:, pl.ds(b0, _BPW)],
                        words_v.at[pl.ds(_C, _NNEG), :])
        pltpu.sync_copy(ivec_hbm.at[pl.ds(b0, _BPW), :], ivec_v)

        @pl.loop(0, _RPW // 16)
        def _(g):
            f = g * 16 + iota16
            bv = f // _R
            rv = f - bv * _R
            vals = plsc.load_gather(words_v, [rv, bv])
            gidx_v[pl.ds(g * 16, 16)] = vals

        def fire(cc, half, sm):
            base = cc * _CROWS
            for q in range(_NG):
                idx = gidx_v.at[pl.ds(base + q * _GCH, _GCH)]
                dst = rbuf_v.at[half, pl.ds(q * _GCH, _GCH), :]
                pltpu.async_copy(ovp_hbm.at[idx], dst, sm)

        def drain(half, sm):
            pltpu.make_async_copy(ovp_hbm.at[pl.ds(0, _CROWS), :],
                                  rbuf_v.at[half], sm).wait()

        def compute(cc, half):
            cbase = pl.multiple_of(cc * _CROWS, 8)

            @pl.loop(0, _CB)
            def _(lb):
                ivr = ivec_v.at[cc * _CB + lb]
                iv0 = ivr[pl.ds(0, 16)]
                iv1 = ivr[pl.ds(16, 16)]
                iv2 = ivr[pl.ds(32, 16)]
                iv3 = ivr[pl.ds(48, 16)]

                @pl.loop(0, _R, step=4)
                def _(r):
                    for u in range(4):
                        row = lb * _R + r + u
                        rr = rbuf_v.at[half, row]
                        acc = (rr[pl.ds(0, 16)] * iv0
                               + rr[pl.ds(16, 16)] * iv1
                               + rr[pl.ds(32, 16)] * iv2
                               + rr[pl.ds(48, 16)] * iv3)
                        tot = jnp.sum(acc)
                        idx16 = jnp.full((16,), cbase + row, jnp.int32)
                        val16 = jnp.full((16,), 0.0, jnp.float32) + tot
                        plsc.store_scatter(sc_v, [idx16], val16, mask=lane0)

        fire(0, 0, sem_a)

        @pl.loop(0, _NCHUNK, step=2)
        def _(c):
            fire(c + 1, 1, sem_b)
            drain(0, sem_a)
            compute(c, 0)

            @pl.when(c + 2 < _NCHUNK)
            def _():
                fire(c + 2, 0, sem_a)

            drain(1, sem_b)
            compute(c + 1, 1)

        pltpu.sync_copy(sc_v, out_hbm.at[pl.ds(r0, _RPW)])

    return body(ovp, ivec, ow_t, nw_t)


def _tc_loss(scores2d):
    rows, cols = scores2d.shape

    def body(s_ref, o_ref):
        s = s_ref[...]
        flat = (lax.broadcasted_iota(jnp.int32, s.shape, 0) * cols
                + lax.broadcasted_iota(jnp.int32, s.shape, 1))
        col40 = lax.rem(flat, _R)
        signed = jnp.where(col40 < _C, s, -s)
        ls = jnp.log(jax.nn.sigmoid(signed))
        o_ref[0, 0] = -jnp.sum(ls) * (1.0 / (_B * _C))

    return pl.pallas_call(
        body,
        out_shape=jax.ShapeDtypeStruct((1, 1), jnp.float32),
        in_specs=[pl.BlockSpec(memory_space=pltpu.VMEM)],
        out_specs=pl.BlockSpec(memory_space=pltpu.SMEM),
    )(scores2d)


def kernel(iwords, owords, nwords, iv_table, ov_table):
    iw = iwords.astype(jnp.int32)
    ow_t = owords.astype(jnp.int32).T
    nw_t = nwords.astype(jnp.int32).T
    ivec = _sc_ivec(iv_table.T, iw)
    ovp = _tc_pack(ov_table.T)
    scores = _sc_scores(ovp, ivec, ow_t, nw_t)
    loss = _tc_loss(scores.reshape(_B * _R // 128, 128))
    return loss[0, 0]


# speedup vs baseline: 1.0350x; 1.0350x over previous
import dataclasses

import jax
import jax.numpy as jnp
from jax import lax
from jax.experimental import pallas as pl
from jax.experimental.pallas import tpu as pltpu
from jax.experimental.pallas import tpu_sc as plsc

_VOCAB = 1000000
_D = 64
_B = 4096
_C = 20
_NNEG = 20
_R = _C + _NNEG
_NW = 32
_BPW = _B // _NW
_RPW = _BPW * _R
_CB = 8
_CROWS = _CB * _R
_GCH = 80
_NG = _CROWS // _GCH
_NCHUNK = _BPW // _CB
_PCB = 16384
_IVR = 8


def _sc_mesh_params():
    mesh = plsc.VectorSubcoreMesh(core_axis_name="c", subcore_axis_name="s")
    cp = pltpu.CompilerParams(use_tc_tiling_on_sc=True)
    if "needs_layout_passes" in pltpu.CompilerParams.__dataclass_fields__:
        cp = dataclasses.replace(cp, needs_layout_passes=False)
    return mesh, cp


def _sc_ivec(iv_t, iwords):
    mesh, cp = _sc_mesh_params()

    @pl.kernel(
        compiler_params=cp,
        out_type=jax.ShapeDtypeStruct((_B, 128), jnp.float32),
        mesh=mesh,
        scratch_types=[
            pltpu.VMEM((_BPW + 16,), jnp.int32),
            pltpu.VMEM((_IVR, _D, 128), jnp.float32),
            pltpu.VMEM((_BPW, 128), jnp.float32),
            pltpu.SemaphoreType.DMA,
        ],
    )
    def body(ivt_hbm, iw_hbm, out_hbm, iw_v, ivblk_v, ivec_v, csem):
        wid = lax.axis_index("s") * 2 + lax.axis_index("c")
        b0 = pl.multiple_of(wid * _BPW, 8)
        iota16 = lax.iota(jnp.int32, 16)
        pltpu.sync_copy(iw_hbm.at[pl.ds(b0, _BPW)], iw_v.at[pl.ds(0, _BPW)])

        def fire(i, slot):
            w = iw_v[pl.ds(i, 16)][0]
            walign = pl.multiple_of((w // 128) * 128, 128)
            pltpu.async_copy(ivt_hbm.at[:, pl.ds(walign, 128)],
                             ivblk_v.at[slot], csem)

        for i in range(_IVR):
            fire(i, i)

        @pl.loop(0, _BPW)
        def _(i):
            slot = lax.rem(i, _IVR)
            pltpu.make_async_copy(ivt_hbm.at[:, pl.ds(0, 128)],
                                  ivblk_v.at[0], csem).wait()
            wv = iw_v[pl.ds(i, 16)][0]
            lane = jnp.full((16,), wv % 128, jnp.int32)
            for j in range(4):
                dv = j * 16 + iota16
                ivec_v[i, pl.ds(j * 16, 16)] = plsc.load_gather(
                    ivblk_v.at[slot], [dv, lane])

            @pl.when(i + _IVR < _BPW)
            def _():
                fire(i + _IVR, slot)

        pltpu.sync_copy(ivec_v, out_hbm.at[pl.ds(b0, _BPW), :])

    return body(iv_t, iwords)


def _tc_pack(ov_t):
    def body(b_ref, ob_ref):
        ipad = jnp.concatenate(
            [jnp.eye(_D, dtype=jnp.float32),
             jnp.zeros((_D, 128 - _D), jnp.float32)], axis=1)
        ob_ref[...] = lax.dot_general(
            b_ref[...], ipad, (((0,), (0,)), ((), ())),
            precision=lax.Precision.DEFAULT)

    return pl.pallas_call(
        body,
        grid=(pl.cdiv(_VOCAB, _PCB),),
        in_specs=[pl.BlockSpec((_D, _PCB), lambda i: (0, i))],
        out_specs=pl.BlockSpec((_PCB, 128), lambda i: (i, 0)),
        out_shape=jax.ShapeDtypeStruct((_VOCAB, 128), jnp.float32),
    )(ov_t)


def _sc_scores(ovp, ivec, ow_t, nw_t):
    mesh, cp = _sc_mesh_params()

    @pl.kernel(
        compiler_params=cp,
        out_type=jax.ShapeDtypeStruct((_B * _R,), jnp.float32),
        mesh=mesh,
        scratch_types=[
            pltpu.VMEM((_BPW, 128), jnp.float32),
            pltpu.VMEM((_R, _BPW), jnp.int32),
            pltpu.VMEM((_RPW,), jnp.int32),
            pltpu.VMEM((2, _CROWS, 128), jnp.float32),
            pltpu.VMEM((_RPW,), jnp.float32),
            pltpu.SemaphoreType.DMA,
            pltpu.SemaphoreType.DMA,
        ],
    )
    def body(ovp_hbm, ivec_hbm, ow_hbm, nw_hbm, out_hbm,
             ivec_v, words_v, gidx_v, rbuf_v, sc_v, sem_a, sem_b):
        wid = lax.axis_index("s") * 2 + lax.axis_index("c")
        b0 = pl.multiple_of(wid * _BPW, 8)
        r0 = pl.multiple_of(wid * _RPW, 8)
        iota16 = lax.iota(jnp.int32, 16)
        lane0 = iota16 == 0
        pltpu.sync_copy(ow_hbm.at[:, pl.ds(b0, _BPW)],
                        words_v.at[pl.ds(0, _C), :])
        pltpu.sync_copy(nw_hbm.at[:, pl.ds(b0, _BPW)],
                        words_v.at[pl.ds(_C, _NNEG), :])
        pltpu.sync_copy(ivec_hbm.at[pl.ds(b0, _BPW), :], ivec_v)

        @pl.loop(0, _RPW // 16)
        def _(g):
            f = g * 16 + iota16
            bv = f // _R
            rv = f - bv * _R
            vals = plsc.load_gather(words_v, [rv, bv])
            gidx_v[pl.ds(g * 16, 16)] = vals

        def fire(cc, half, sm):
            base = cc * _CROWS
            for q in range(_NG):
                idx = gidx_v.at[pl.ds(base + q * _GCH, _GCH)]
                dst = rbuf_v.at[half, pl.ds(q * _GCH, _GCH), :]
                pltpu.async_copy(ovp_hbm.at[idx], dst, sm)

        def drain(half, sm):
            pltpu.make_async_copy(ovp_hbm.at[pl.ds(0, _CROWS), :],
                                  rbuf_v.at[half], sm).wait()

        def compute(cc, half):
            cbase = pl.multiple_of(cc * _CROWS, 8)

            @pl.loop(0, _CB)
            def _(lb):
                ivr = ivec_v.at[cc * _CB + lb]
                iv0 = ivr[pl.ds(0, 16)]
                iv1 = ivr[pl.ds(16, 16)]
                iv2 = ivr[pl.ds(32, 16)]
                iv3 = ivr[pl.ds(48, 16)]

                @pl.loop(0, _R, step=4)
                def _(r):
                    for u in range(4):
                        row = lb * _R + r + u
                        rr = rbuf_v.at[half, row]
                        acc = (rr[pl.ds(0, 16)] * iv0
                               + rr[pl.ds(16, 16)] * iv1
                               + rr[pl.ds(32, 16)] * iv2
                               + rr[pl.ds(48, 16)] * iv3)
                        tot = jnp.sum(acc)
                        idx16 = jnp.full((16,), cbase + row, jnp.int32)
                        val16 = jnp.full((16,), 0.0, jnp.float32) + tot
                        plsc.store_scatter(sc_v, [idx16], val16, mask=lane0)

        fire(0, 0, sem_a)

        @pl.loop(0, _NCHUNK, step=2)
        def _(c):
            fire(c + 1, 1, sem_b)
            drain(0, sem_a)
            compute(c, 0)

            @pl.when(c + 2 < _NCHUNK)
            def _():
                fire(c + 2, 0, sem_a)

            drain(1, sem_b)
            compute(c + 1, 1)

        pltpu.sync_copy(sc_v, out_hbm.at[pl.ds(r0, _RPW)])

    return body(ovp, ivec, ow_t, nw_t)


def _tc_loss(scores2d):
    rows, cols = scores2d.shape

    def body(s_ref, o_ref):
        s = s_ref[...]
        flat = (lax.broadcasted_iota(jnp.int32, s.shape, 0) * cols
                + lax.broadcasted_iota(jnp.int32, s.shape, 1))
        col40 = lax.rem(flat, _R)
        signed = jnp.where(col40 < _C, s, -s)
        ls = jnp.log(jax.nn.sigmoid(signed))
        o_ref[0, 0] = -jnp.sum(ls) * (1.0 / (_B * _C))

    return pl.pallas_call(
        body,
        out_shape=jax.ShapeDtypeStruct((1, 1), jnp.float32),
        in_specs=[pl.BlockSpec(memory_space=pltpu.VMEM)],
        out_specs=pl.BlockSpec(memory_space=pltpu.SMEM),
    )(scores2d)


def kernel(iwords, owords, nwords, iv_table, ov_table):
    iw = iwords.astype(jnp.int32)
    ow_t = owords.astype(jnp.int32).T
    nw_t = nwords.astype(jnp.int32).T
    ivec = _sc_ivec(iv_table.T, iw)
    ovp = _tc_pack(ov_table.T)
    scores = _sc_scores(ovp, ivec, ow_t, nw_t)
    loss = _tc_loss(scores.reshape(_B * _R // 128, 128))
    return loss[0, 0]


# speedup vs baseline: 1.0535x; 1.0179x over previous
import dataclasses

import jax
import jax.numpy as jnp
from jax import lax
from jax.experimental import pallas as pl
from jax.experimental.pallas import tpu as pltpu
from jax.experimental.pallas import tpu_sc as plsc

_VOCAB = 1000000
_D = 64
_B = 4096
_C = 20
_NNEG = 20
_R = _C + _NNEG
_NW = 32
_BPW = _B // _NW
_RPW = _BPW * _R
_CB = 8
_CROWS = _CB * _R
_GCH = 80
_NG = _CROWS // _GCH
_NCHUNK = _BPW // _CB
_PCB = 32768
_IVR = 8


def _sc_mesh_params():
    mesh = plsc.VectorSubcoreMesh(core_axis_name="c", subcore_axis_name="s")
    cp = pltpu.CompilerParams(use_tc_tiling_on_sc=True)
    if "needs_layout_passes" in pltpu.CompilerParams.__dataclass_fields__:
        cp = dataclasses.replace(cp, needs_layout_passes=False)
    return mesh, cp


def _sc_ivec(iv_t, iwords):
    mesh, cp = _sc_mesh_params()

    @pl.kernel(
        compiler_params=cp,
        out_type=jax.ShapeDtypeStruct((_B, 128), jnp.float32),
        mesh=mesh,
        scratch_types=[
            pltpu.VMEM((_BPW + 16,), jnp.int32),
            pltpu.VMEM((_IVR, _D, 128), jnp.float32),
            pltpu.VMEM((_BPW, 128), jnp.float32),
            pltpu.SemaphoreType.DMA,
        ],
    )
    def body(ivt_hbm, iw_hbm, out_hbm, iw_v, ivblk_v, ivec_v, csem):
        wid = lax.axis_index("s") * 2 + lax.axis_index("c")
        b0 = pl.multiple_of(wid * _BPW, 8)
        iota16 = lax.iota(jnp.int32, 16)
        pltpu.sync_copy(iw_hbm.at[pl.ds(b0, _BPW)], iw_v.at[pl.ds(0, _BPW)])

        def fire(i, slot):
            w = iw_v[pl.ds(i, 16)][0]
            walign = pl.multiple_of((w // 128) * 128, 128)
            pltpu.async_copy(ivt_hbm.at[:, pl.ds(walign, 128)],
                             ivblk_v.at[slot], csem)

        for i in range(_IVR):
            fire(i, i)

        @pl.loop(0, _BPW)
        def _(i):
            slot = lax.rem(i, _IVR)
            pltpu.make_async_copy(ivt_hbm.at[:, pl.ds(0, 128)],
                                  ivblk_v.at[0], csem).wait()
            wv = iw_v[pl.ds(i, 16)][0]
            lane = jnp.full((16,), wv % 128, jnp.int32)
            for j in range(4):
                dv = j * 16 + iota16
                ivec_v[i, pl.ds(j * 16, 16)] = plsc.load_gather(
                    ivblk_v.at[slot], [dv, lane])

            @pl.when(i + _IVR < _BPW)
            def _():
                fire(i + _IVR, slot)

        pltpu.sync_copy(ivec_v, out_hbm.at[pl.ds(b0, _BPW), :])

    return body(iv_t, iwords)


def _tc_pack(ov_t):
    def body(b_ref, ob_ref):
        ipad = jnp.concatenate(
            [jnp.eye(_D, dtype=jnp.float32),
             jnp.zeros((_D, 128 - _D), jnp.float32)], axis=1)
        ob_ref[...] = lax.dot_general(
            b_ref[...], ipad, (((0,), (0,)), ((), ())),
            precision=lax.Precision.DEFAULT)

    return pl.pallas_call(
        body,
        grid=(pl.cdiv(_VOCAB, _PCB),),
        in_specs=[pl.BlockSpec((_D, _PCB), lambda i: (0, i))],
        out_specs=pl.BlockSpec((_PCB, 128), lambda i: (i, 0)),
        out_shape=jax.ShapeDtypeStruct((_VOCAB, 128), jnp.float32),
    )(ov_t)


def _sc_scores(ovp, ivec, ow_t, nw_t):
    mesh, cp = _sc_mesh_params()

    @pl.kernel(
        compiler_params=cp,
        out_type=jax.ShapeDtypeStruct((_B * _R,), jnp.float32),
        mesh=mesh,
        scratch_types=[
            pltpu.VMEM((_BPW, 128), jnp.float32),
            pltpu.VMEM((_R, _BPW), jnp.int32),
            pltpu.VMEM((_RPW,), jnp.int32),
            pltpu.VMEM((2, _CROWS, 128), jnp.float32),
            pltpu.VMEM((_RPW,), jnp.float32),
            pltpu.SemaphoreType.DMA,
            pltpu.SemaphoreType.DMA,
        ],
    )
    def body(ovp_hbm, ivec_hbm, ow_hbm, nw_hbm, out_hbm,
             ivec_v, words_v, gidx_v, rbuf_v, sc_v, sem_a, sem_b):
        wid = lax.axis_index("s") * 2 + lax.axis_index("c")
        b0 = pl.multiple_of(wid * _BPW, 8)
        r0 = pl.multiple_of(wid * _RPW, 8)
        iota16 = lax.iota(jnp.int32, 16)
        lane0 = iota16 == 0
        pltpu.sync_copy(ow_hbm.at[:, pl.ds(b0, _BPW)],
                        words_v.at[pl.ds(0, _C), :])
        pltpu.sync_copy(nw_hbm.at[:, pl.ds(b0, _BPW)],
                        words_v.at[pl.ds(_C, _NNEG), :])
        pltpu.sync_copy(ivec_hbm.at[pl.ds(b0, _BPW), :], ivec_v)

        @pl.loop(0, _RPW // 16)
        def _(g):
            f = g * 16 + iota16
            bv = f // _R
            rv = f - bv * _R
            vals = plsc.load_gather(words_v, [rv, bv])
            gidx_v[pl.ds(g * 16, 16)] = vals

        def fire(cc, half, sm):
            base = cc * _CROWS
            for q in range(_NG):
                idx = gidx_v.at[pl.ds(base + q * _GCH, _GCH)]
                dst = rbuf_v.at[half, pl.ds(q * _GCH, _GCH), :]
                pltpu.async_copy(ovp_hbm.at[idx], dst, sm)

        def drain(half, sm):
            pltpu.make_async_copy(ovp_hbm.at[pl.ds(0, _CROWS), :],
                                  rbuf_v.at[half], sm).wait()

        def compute(cc, half):
            cbase = pl.multiple_of(cc * _CROWS, 8)

            @pl.loop(0, _CB)
            def _(lb):
                ivr = ivec_v.at[cc * _CB + lb]
                iv0 = ivr[pl.ds(0, 16)]
                iv1 = ivr[pl.ds(16, 16)]
                iv2 = ivr[pl.ds(32, 16)]
                iv3 = ivr[pl.ds(48, 16)]

                @pl.loop(0, _R, step=4)
                def _(r):
                    for u in range(4):
                        row = lb * _R + r + u
                        rr = rbuf_v.at[half, row]
                        acc = (rr[pl.ds(0, 16)] * iv0
                               + rr[pl.ds(16, 16)] * iv1
                               + rr[pl.ds(32, 16)] * iv2
                               + rr[pl.ds(48, 16)] * iv3)
                        tot = jnp.sum(acc)
                        idx16 = jnp.full((16,), cbase + row, jnp.int32)
                        val16 = jnp.full((16,), 0.0, jnp.float32) + tot
                        plsc.store_scatter(sc_v, [idx16], val16, mask=lane0)

        fire(0, 0, sem_a)

        @pl.loop(0, _NCHUNK, step=2)
        def _(c):
            fire(c + 1, 1, sem_b)
            drain(0, sem_a)
            compute(c, 0)

            @pl.when(c + 2 < _NCHUNK)
            def _():
                fire(c + 2, 0, sem_a)

            drain(1, sem_b)
            compute(c + 1, 1)

        pltpu.sync_copy(sc_v, out_hbm.at[pl.ds(r0, _RPW)])

    return body(ovp, ivec, ow_t, nw_t)


def _tc_loss(scores2d):
    rows, cols = scores2d.shape

    def body(s_ref, o_ref):
        s = s_ref[...]
        flat = (lax.broadcasted_iota(jnp.int32, s.shape, 0) * cols
                + lax.broadcasted_iota(jnp.int32, s.shape, 1))
        col40 = lax.rem(flat, _R)
        signed = jnp.where(col40 < _C, s, -s)
        ls = jnp.log(jax.nn.sigmoid(signed))
        o_ref[0, 0] = -jnp.sum(ls) * (1.0 / (_B * _C))

    return pl.pallas_call(
        body,
        out_shape=jax.ShapeDtypeStruct((1, 1), jnp.float32),
        in_specs=[pl.BlockSpec(memory_space=pltpu.VMEM)],
        out_specs=pl.BlockSpec(memory_space=pltpu.SMEM),
    )(scores2d)


def kernel(iwords, owords, nwords, iv_table, ov_table):
    iw = iwords.astype(jnp.int32)
    ow_t = owords.astype(jnp.int32).T
    nw_t = nwords.astype(jnp.int32).T
    ivec = _sc_ivec(iv_table.T, iw)
    ovp = _tc_pack(ov_table.T)
    scores = _sc_scores(ovp, ivec, ow_t, nw_t)
    loss = _tc_loss(scores.reshape(_B * _R // 128, 128))
    return loss[0, 0]


# speedup vs baseline: 1.0538x; 1.0003x over previous
import dataclasses

import jax
import jax.numpy as jnp
from jax import lax
from jax.experimental import pallas as pl
from jax.experimental.pallas import tpu as pltpu
from jax.experimental.pallas import tpu_sc as plsc

_VOCAB = 1000000
_D = 64
_B = 4096
_C = 20
_NNEG = 20
_R = _C + _NNEG
_NW = 32
_BPW = _B // _NW
_RPW = _BPW * _R
_CB = 8
_CROWS = _CB * _R
_GCH = 80
_NG = _CROWS // _GCH
_NCHUNK = _BPW // _CB
_PCB = 32768
_IVR = 8


def _sc_mesh_params():
    mesh = plsc.VectorSubcoreMesh(core_axis_name="c", subcore_axis_name="s")
    cp = pltpu.CompilerParams(use_tc_tiling_on_sc=True)
    if "needs_layout_passes" in pltpu.CompilerParams.__dataclass_fields__:
        cp = dataclasses.replace(cp, needs_layout_passes=False)
    return mesh, cp


def _sc_ivec(iv_t, iwords):
    mesh, cp = _sc_mesh_params()

    @pl.kernel(
        compiler_params=cp,
        out_type=jax.ShapeDtypeStruct((_B, 128), jnp.float32),
        mesh=mesh,
        scratch_types=[
            pltpu.VMEM((_BPW + 16,), jnp.int32),
            pltpu.VMEM((_IVR, _D, 128), jnp.float32),
            pltpu.VMEM((_BPW, 128), jnp.float32),
            pltpu.SemaphoreType.DMA,
        ],
    )
    def body(ivt_hbm, iw_hbm, out_hbm, iw_v, ivblk_v, ivec_v, csem):
        wid = lax.axis_index("s") * 2 + lax.axis_index("c")
        b0 = pl.multiple_of(wid * _BPW, 8)
        iota16 = lax.iota(jnp.int32, 16)
        pltpu.sync_copy(iw_hbm.at[pl.ds(b0, _BPW)], iw_v.at[pl.ds(0, _BPW)])

        def fire(i, slot):
            w = iw_v[pl.ds(i, 16)][0]
            walign = pl.multiple_of((w // 128) * 128, 128)
            pltpu.async_copy(ivt_hbm.at[:, pl.ds(walign, 128)],
                             ivblk_v.at[slot], csem)

        for i in range(_IVR):
            fire(i, i)

        @pl.loop(0, _BPW)
        def _(i):
            slot = lax.rem(i, _IVR)
            pltpu.make_async_copy(ivt_hbm.at[:, pl.ds(0, 128)],
                                  ivblk_v.at[0], csem).wait()
            wv = iw_v[pl.ds(i, 16)][0]
            lane = jnp.full((16,), wv % 128, jnp.int32)
            for j in range(4):
                dv = j * 16 + iota16
                ivec_v[i, pl.ds(j * 16, 16)] = plsc.load_gather(
                    ivblk_v.at[slot], [dv, lane])

            @pl.when(i + _IVR < _BPW)
            def _():
                fire(i + _IVR, slot)

        pltpu.sync_copy(ivec_v, out_hbm.at[pl.ds(b0, _BPW), :])

    return body(iv_t, iwords)


def _tc_pack(ov_t):
    def body(b_ref, ob_ref):
        ipad = jnp.concatenate(
            [jnp.eye(_D, dtype=jnp.float32),
             jnp.zeros((_D, 128 - _D), jnp.float32)], axis=1)
        ob_ref[...] = lax.dot_general(
            b_ref[...], ipad, (((0,), (0,)), ((), ())),
            precision=lax.Precision.DEFAULT)

    return pl.pallas_call(
        body,
        grid=(pl.cdiv(_VOCAB, _PCB),),
        in_specs=[pl.BlockSpec((_D, _PCB), lambda i: (0, i))],
        out_specs=pl.BlockSpec((_PCB, 128), lambda i: (i, 0)),
        out_shape=jax.ShapeDtypeStruct((_VOCAB, 128), jnp.float32),
    )(ov_t)


def _sc_scores(ovp, ivec, ow_t, nw_t):
    mesh, cp = _sc_mesh_params()

    @pl.kernel(
        compiler_params=cp,
        out_type=jax.ShapeDtypeStruct((_B * _R,), jnp.float32),
        mesh=mesh,
        scratch_types=[
            pltpu.VMEM((_BPW, 128), jnp.float32),
            pltpu.VMEM((_R, _BPW), jnp.int32),
            pltpu.VMEM((_RPW,), jnp.int32),
            pltpu.VMEM((2, _CROWS, 128), jnp.float32),
            pltpu.VMEM((_RPW,), jnp.float32),
            pltpu.SemaphoreType.DMA,
            pltpu.SemaphoreType.DMA,
        ],
    )
    def body(ovp_hbm, ivec_hbm, ow_hbm, nw_hbm, out_hbm,
             ivec_v, words_v, gidx_v, rbuf_v, sc_v, sem_a, sem_b):
        wid = lax.axis_index("s") * 2 + lax.axis_index("c")
        b0 = pl.multiple_of(wid * _BPW, 8)
        r0 = pl.multiple_of(wid * _RPW, 8)
        iota16 = lax.iota(jnp.int32, 16)
        lane0 = iota16 == 0
        pltpu.sync_copy(ow_hbm.at[:, pl.ds(b0, _BPW)],
                        words_v.at[pl.ds(0, _C), :])
        pltpu.sync_copy(nw_hbm.at[:, pl.ds(b0, _BPW)],
                        words_v.at[pl.ds(_C, _NNEG), :])
        pltpu.sync_copy(ivec_hbm.at[pl.ds(b0, _BPW), :], ivec_v)

        @pl.loop(0, _RPW // 16)
        def _(g):
            f = g * 16 + iota16
            bv = f // _R
            rv = f - bv * _R
            vals = plsc.load_gather(words_v, [rv, bv])
            gidx_v[pl.ds(g * 16, 16)] = vals

        def fire(cc, half, sm):
            base = cc * _CROWS
            for q in range(_NG):
                idx = gidx_v.at[pl.ds(base + q * _GCH, _GCH)]
                dst = rbuf_v.at[half, pl.ds(q * _GCH, _GCH), :]
                pltpu.async_copy(ovp_hbm.at[idx], dst, sm)

        def drain(half, sm):
            pltpu.make_async_copy(ovp_hbm.at[pl.ds(0, _CROWS), :],
                                  rbuf_v.at[half], sm).wait()

        def compute(cc, half):
            cbase = pl.multiple_of(cc * _CROWS, 8)

            @pl.loop(0, _CB)
            def _(lb):
                ivr = ivec_v.at[cc * _CB + lb]
                iv0 = ivr[pl.ds(0, 16)]
                iv1 = ivr[pl.ds(16, 16)]
                iv2 = ivr[pl.ds(32, 16)]
                iv3 = ivr[pl.ds(48, 16)]

                @pl.loop(0, _R, step=8)
                def _(r):
                    for u in range(8):
                        row = lb * _R + r + u
                        rr = rbuf_v.at[half, row]
                        acc = (rr[pl.ds(0, 16)] * iv0
                               + rr[pl.ds(16, 16)] * iv1
                               + rr[pl.ds(32, 16)] * iv2
                               + rr[pl.ds(48, 16)] * iv3)
                        tot = jnp.sum(acc)
                        idx16 = jnp.full((16,), cbase + row, jnp.int32)
                        val16 = jnp.full((16,), 0.0, jnp.float32) + tot
                        plsc.store_scatter(sc_v, [idx16], val16, mask=lane0)

        fire(0, 0, sem_a)

        @pl.loop(0, _NCHUNK, step=2)
        def _(c):
            fire(c + 1, 1, sem_b)
            drain(0, sem_a)
            compute(c, 0)

            @pl.when(c + 2 < _NCHUNK)
            def _():
                fire(c + 2, 0, sem_a)

            drain(1, sem_b)
            compute(c + 1, 1)

        pltpu.sync_copy(sc_v, out_hbm.at[pl.ds(r0, _RPW)])

    return body(ovp, ivec, ow_t, nw_t)


def _tc_loss(scores2d):
    rows, cols = scores2d.shape

    def body(s_ref, o_ref):
        s = s_ref[...]
        flat = (lax.broadcasted_iota(jnp.int32, s.shape, 0) * cols
                + lax.broadcasted_iota(jnp.int32, s.shape, 1))
        col40 = lax.rem(flat, _R)
        signed = jnp.where(col40 < _C, s, -s)
        ls = jnp.log(jax.nn.sigmoid(signed))
        o_ref[0, 0] = -jnp.sum(ls) * (1.0 / (_B * _C))

    return pl.pallas_call(
        body,
        out_shape=jax.ShapeDtypeStruct((1, 1), jnp.float32),
        in_specs=[pl.BlockSpec(memory_space=pltpu.VMEM)],
        out_specs=pl.BlockSpec(memory_space=pltpu.SMEM),
    )(scores2d)


def kernel(iwords, owords, nwords, iv_table, ov_table):
    iw = iwords.astype(jnp.int32)
    ow_t = owords.astype(jnp.int32).T
    nw_t = nwords.astype(jnp.int32).T
    ivec = _sc_ivec(iv_table.T, iw)
    ovp = _tc_pack(ov_table.T)
    scores = _sc_scores(ovp, ivec, ow_t, nw_t)
    loss = _tc_loss(scores.reshape(_B * _R // 128, 128))
    return loss[0, 0]
